# Initial kernel scaffold; baseline (speedup 1.0000x reference)
import jax
import jax.numpy as jnp
from jax.experimental import pallas as pl


def kernel(feat, efeat, W_msg, b_msg, W_attn, b_attn, edge_index):
    raise NotImplementedError("write your pallas kernel here")


# speedup vs baseline: 28.5974x; 28.5974x over previous
import functools

import jax
import jax.numpy as jnp
from jax import lax
from jax.experimental import pallas as pl
from jax.experimental.pallas import tpu as pltpu
from jax.experimental.pallas import tpu_sc as plsc

N_NODES = 10000
N_EDGES = 160000
IN_F = 8
OUT_F = 8
D_EDGE = 16
CH = IN_F * OUT_F
NW = 32
NPT = 313
LAST_CNT = N_NODES - (NW - 1) * NPT
CHUNK = 2000
NCHUNK = N_EDGES // CHUNK
NEG = -3.0e38


def _mm_body(ef, wm, bm, wa, ba, om, oa):
    x = ef[...]
    om[...] = jnp.dot(x, wm[...], preferred_element_type=jnp.float32) + bm[...]
    oa[...] = jnp.dot(x, wa[...], preferred_element_type=jnp.float32) + ba[...]


def _edge_mats(efeat, W_msg, b_msg, W_attn, b_attn):
    BLK = 2000
    return pl.pallas_call(
        _mm_body,
        grid=(N_EDGES // BLK,),
        in_specs=[
            pl.BlockSpec((BLK, D_EDGE), lambda i: (i, 0)),
            pl.BlockSpec((D_EDGE, CH), lambda i: (0, 0)),
            pl.BlockSpec((1, CH), lambda i: (0, 0)),
            pl.BlockSpec((D_EDGE, CH), lambda i: (0, 0)),
            pl.BlockSpec((1, CH), lambda i: (0, 0)),
        ],
        out_specs=[
            pl.BlockSpec((BLK, CH), lambda i: (i, 0)),
            pl.BlockSpec((BLK, CH), lambda i: (i, 0)),
        ],
        out_shape=[
            jax.ShapeDtypeStruct((N_EDGES, CH), jnp.float32),
            jax.ShapeDtypeStruct((N_EDGES, CH), jnp.float32),
        ],
    )(efeat, W_msg, b_msg.reshape(1, CH), W_attn, b_attn.reshape(1, CH))


def _sc_body(wa_hbm, wm_hbm, featp_hbm, src_hbm, dst_hbm, out_hbm,
             amax, aden, anum, outb, dstc, srcc, dstb, srcb, eidb,
             wa16, wm16, h16, sem_a, sem_m, sem_h):
    wid = lax.axis_index("s") * 2 + lax.axis_index("c")
    lo = wid * NPT
    cnt = jnp.minimum(N_NODES - lo, NPT)
    iota = lax.iota(jnp.int32, 16)
    lane8 = (iota >= 8).astype(jnp.int32)
    io7 = iota & 7
    negv = jnp.full((16,), NEG, jnp.float32)
    zerov = jnp.zeros((16,), jnp.float32)

    def init_acc(i, _):
        s = pl.ds(i * 16, 16)
        amax[s] = negv
        aden[s] = zerov
        anum[s] = zerov
        return 0

    lax.fori_loop(0, (NPT + 1) * CH // 16, init_acc, 0)

    def init_out(i, _):
        outb[pl.ds(i * 16, 16)] = zerov
        return 0

    lax.fori_loop(0, (NPT * OUT_F + 32) // 16, init_out, 0)

    def fetch(b, want_wm):
        eb = eidb[pl.ds(b * 16, 16)]
        sb = srcb[pl.ds(b * 16, 16)]
        cpa = pltpu.async_copy(wa_hbm.at[eb], wa16, sem_a)
        cph = pltpu.async_copy(featp_hbm.at[sb], h16, sem_h)
        if want_wm:
            cpm = pltpu.async_copy(wm_hbm.at[eb], wm16, sem_m)
        cpa.wait()
        cph.wait()
        if want_wm:
            cpm.wait()

    def mb1(b, _):
        fetch(b, False)

        def edge1(j, _):
            dl = plsc.load_gather(dstb, [jnp.full((16,), b * 16 + j, jnp.int32)])
            base64 = dl * CH
            js = jnp.full((16,), j, jnp.int32)
            for k in range(4):
                hk = plsc.load_gather(h16, [js, lane8 + 2 * k])
                wak = plsc.load_gather(wa16, [js, iota + 16 * k])
                idxk = base64 + (iota + 16 * k)
                mk = plsc.load_gather(amax, [idxk])
                plsc.store_scatter(amax, [idxk], jnp.maximum(mk, wak * hk))
            return 0

        lax.fori_loop(0, 16, edge1, 0)
        return 0

    def mb2(b, _):
        fetch(b, True)

        def edge2(j, _):
            dl = plsc.load_gather(dstb, [jnp.full((16,), b * 16 + j, jnp.int32)])
            base64 = dl * CH
            js = jnp.full((16,), j, jnp.int32)
            for k in range(4):
                hk = plsc.load_gather(h16, [js, lane8 + 2 * k])
                wak = plsc.load_gather(wa16, [js, iota + 16 * k])
                wmk = plsc.load_gather(wm16, [js, iota + 16 * k])
                idxk = base64 + (iota + 16 * k)
                mk = plsc.load_gather(amax, [idxk])
                p = jnp.exp(wak * hk - mk)
                plsc.addupdate_scatter(aden, [idxk], p)
                plsc.addupdate_scatter(anum, [idxk], wmk * hk * p)
            return 0

        lax.fori_loop(0, 16, edge2, 0)
        return 0

    def run_pass(edge_mb):
        def chunk_body(ci, _):
            base = ci * CHUNK
            pltpu.sync_copy(dst_hbm.at[pl.ds(base, CHUNK)], dstc)
            pltpu.sync_copy(src_hbm.at[pl.ds(base, CHUNK)], srcc)

            def bin_body(v, off):
                d = dstc[pl.ds(v * 16, 16)]
                m = (d >= lo) & (d < lo + cnt)
                mi = m.astype(jnp.int32)
                pos = off + plsc.cumsum(mi) - 1
                plsc.store_scatter(dstb, [pos], d - lo, mask=m)
                plsc.store_scatter(srcb, [pos], srcc[pl.ds(v * 16, 16)], mask=m)
                plsc.store_scatter(eidb, [pos], iota + (base + v * 16), mask=m)
                return off + jnp.sum(mi)

            n = lax.fori_loop(0, CHUNK // 16, bin_body, jnp.int32(0))
            tl = pl.ds(n, 16)
            dstb[tl] = jnp.full((16,), NPT, jnp.int32)
            srcb[tl] = jnp.zeros((16,), jnp.int32)
            eidb[tl] = jnp.zeros((16,), jnp.int32)
            lax.fori_loop(0, (n + 15) // 16, edge_mb, 0)
            return 0

        lax.fori_loop(0, NCHUNK, chunk_body, 0)

    run_pass(mb1)
    run_pass(mb2)

    def fin(nn, _):
        b64 = nn * CH
        acc = zerov
        for k in range(4):
            s = pl.ds(b64 + 16 * k, 16)
            acc = acc + anum[s] / jnp.maximum(aden[s], 1.0)
        oidx = nn * OUT_F + io7
        plsc.addupdate_scatter(outb, [oidx], acc, mask=iota < 8)
        plsc.addupdate_scatter(outb, [oidx], acc, mask=iota >= 8)
        return 0

    lax.fori_loop(0, cnt, fin, 0)

    @pl.when(wid < NW - 1)
    def _():
        pltpu.sync_copy(outb.at[pl.ds(0, NPT * OUT_F)],
                        out_hbm.at[pl.ds(lo * OUT_F, NPT * OUT_F)])

    @pl.when(wid == NW - 1)
    def _():
        pltpu.sync_copy(outb.at[pl.ds(0, LAST_CNT * OUT_F)],
                        out_hbm.at[pl.ds(lo * OUT_F, LAST_CNT * OUT_F)])


def _sc_call(wa, wm, featp, src, dst):
    kern = pl.kernel(
        _sc_body,
        out_type=jax.ShapeDtypeStruct((N_NODES * OUT_F,), jnp.float32),
        mesh=plsc.VectorSubcoreMesh(core_axis_name="c", subcore_axis_name="s",
                                    num_cores=2, num_subcores=16),
        scratch_types=[
            pltpu.VMEM(((NPT + 1) * CH,), jnp.float32),
            pltpu.VMEM(((NPT + 1) * CH,), jnp.float32),
            pltpu.VMEM(((NPT + 1) * CH,), jnp.float32),
            pltpu.VMEM((NPT * OUT_F + 32,), jnp.float32),
            pltpu.VMEM((CHUNK,), jnp.int32),
            pltpu.VMEM((CHUNK,), jnp.int32),
            pltpu.VMEM((CHUNK + 16,), jnp.int32),
            pltpu.VMEM((CHUNK + 16,), jnp.int32),
            pltpu.VMEM((CHUNK + 16,), jnp.int32),
            pltpu.VMEM((16, CH), jnp.float32),
            pltpu.VMEM((16, CH), jnp.float32),
            pltpu.VMEM((16, D_EDGE), jnp.float32),
            pltpu.SemaphoreType.DMA,
            pltpu.SemaphoreType.DMA,
            pltpu.SemaphoreType.DMA,
        ],
        compiler_params=pltpu.CompilerParams(
            needs_layout_passes=False, use_tc_tiling_on_sc=False),
    )
    return kern(wa, wm, featp, src, dst)


def kernel(feat, efeat, W_msg, b_msg, W_attn, b_attn, edge_index):
    wm, wa = _edge_mats(efeat, W_msg, b_msg, W_attn, b_attn)
    featp = jnp.pad(feat, ((0, 0), (0, D_EDGE - IN_F)))
    src = edge_index[0].astype(jnp.int32)
    dst = edge_index[1].astype(jnp.int32)
    out = _sc_call(wa, wm, featp, src, dst)
    return out.reshape(N_NODES, OUT_F)


# speedup vs baseline: 28.8106x; 1.0075x over previous
import functools

import jax
import jax.numpy as jnp
from jax import lax
from jax.experimental import pallas as pl
from jax.experimental.pallas import tpu as pltpu
from jax.experimental.pallas import tpu_sc as plsc

N_NODES = 10000
N_EDGES = 160000
IN_F = 8
OUT_F = 8
D_EDGE = 16
CH = IN_F * OUT_F
NW = 32
NPT = 313
LAST_CNT = N_NODES - (NW - 1) * NPT
CHUNK = 2000
NCHUNK = N_EDGES // CHUNK
NEG = -3.0e38


def _mm_body(ef, wm, bm, wa, ba, om, oa):
    x = ef[...]
    om[...] = jnp.dot(x, wm[...], preferred_element_type=jnp.float32) + bm[...]
    oa[...] = jnp.dot(x, wa[...], preferred_element_type=jnp.float32) + ba[...]


def _edge_mats(efeat, W_msg, b_msg, W_attn, b_attn):
    BLK = 2000
    return pl.pallas_call(
        _mm_body,
        grid=(N_EDGES // BLK,),
        in_specs=[
            pl.BlockSpec((BLK, D_EDGE), lambda i: (i, 0)),
            pl.BlockSpec((D_EDGE, CH), lambda i: (0, 0)),
            pl.BlockSpec((1, CH), lambda i: (0, 0)),
            pl.BlockSpec((D_EDGE, CH), lambda i: (0, 0)),
            pl.BlockSpec((1, CH), lambda i: (0, 0)),
        ],
        out_specs=[
            pl.BlockSpec((BLK, CH), lambda i: (i, 0)),
            pl.BlockSpec((BLK, CH), lambda i: (i, 0)),
        ],
        out_shape=[
            jax.ShapeDtypeStruct((N_EDGES, CH), jnp.float32),
            jax.ShapeDtypeStruct((N_EDGES, CH), jnp.float32),
        ],
    )(efeat, W_msg, b_msg.reshape(1, CH), W_attn, b_attn.reshape(1, CH))


def _sc_body(wa_hbm, wm_hbm, featp_hbm, src_hbm, dst_hbm, out_hbm,
             amax, aden, anum, outb, dstc, srcc, dstb, srcb, eidb,
             wa16, wm16, h16, sem_a, sem_m, sem_h):
    wid = lax.axis_index("s") * 2 + lax.axis_index("c")
    lo = wid * NPT
    cnt = jnp.minimum(N_NODES - lo, NPT)
    iota = lax.iota(jnp.int32, 16)
    lane8 = (iota >= 8).astype(jnp.int32)
    io7 = iota & 7
    negv = jnp.full((16,), NEG, jnp.float32)
    zerov = jnp.zeros((16,), jnp.float32)

    def init_acc(i, _):
        s = pl.ds(i * 16, 16)
        amax[s] = negv
        aden[s] = zerov
        anum[s] = zerov
        return 0

    lax.fori_loop(0, (NPT + 1) * CH // 16, init_acc, 0)

    def init_out(i, _):
        outb[pl.ds(i * 16, 16)] = zerov
        return 0

    lax.fori_loop(0, (NPT * OUT_F + 32) // 16, init_out, 0)

    def fetch(b, want_wm):
        eb = eidb[pl.ds(b * 16, 16)]
        sb = srcb[pl.ds(b * 16, 16)]
        cpa = pltpu.async_copy(wa_hbm.at[eb], wa16, sem_a)
        cph = pltpu.async_copy(featp_hbm.at[sb], h16, sem_h)
        if want_wm:
            cpm = pltpu.async_copy(wm_hbm.at[eb], wm16, sem_m)
        cpa.wait()
        cph.wait()
        if want_wm:
            cpm.wait()

    def mb1(b, _):
        fetch(b, False)

        def edge1(j, _):
            dl = plsc.load_gather(dstb, [jnp.full((16,), b * 16 + j, jnp.int32)])
            base64 = dl * CH
            js = jnp.full((16,), j, jnp.int32)
            for k in range(4):
                hk = plsc.load_gather(h16, [js, lane8 + 2 * k])
                wak = plsc.load_gather(wa16, [js, iota + 16 * k])
                idxk = base64 + (iota + 16 * k)
                mk = plsc.load_gather(amax, [idxk])
                plsc.store_scatter(amax, [idxk], jnp.maximum(mk, wak * hk))
            return 0

        lax.fori_loop(0, 16, edge1, 0, unroll=8)
        return 0

    def mb2(b, _):
        fetch(b, True)

        def edge2(j, _):
            dl = plsc.load_gather(dstb, [jnp.full((16,), b * 16 + j, jnp.int32)])
            base64 = dl * CH
            js = jnp.full((16,), j, jnp.int32)
            for k in range(4):
                hk = plsc.load_gather(h16, [js, lane8 + 2 * k])
                wak = plsc.load_gather(wa16, [js, iota + 16 * k])
                wmk = plsc.load_gather(wm16, [js, iota + 16 * k])
                idxk = base64 + (iota + 16 * k)
                mk = plsc.load_gather(amax, [idxk])
                p = jnp.exp(wak * hk - mk)
                plsc.addupdate_scatter(aden, [idxk], p)
                plsc.addupdate_scatter(anum, [idxk], wmk * hk * p)
            return 0

        lax.fori_loop(0, 16, edge2, 0, unroll=8)
        return 0

    def run_pass(edge_mb):
        def chunk_body(ci, _):
            base = ci * CHUNK
            pltpu.sync_copy(dst_hbm.at[pl.ds(base, CHUNK)], dstc)
            pltpu.sync_copy(src_hbm.at[pl.ds(base, CHUNK)], srcc)

            def bin_body(v, off):
                d = dstc[pl.ds(v * 16, 16)]
                m = (d >= lo) & (d < lo + cnt)
                cum = plsc.cumsum(m.astype(jnp.int32))
                pos = off + cum - 1
                plsc.store_scatter(dstb, [pos], d - lo, mask=m)
                plsc.store_scatter(srcb, [pos], srcc[pl.ds(v * 16, 16)], mask=m)
                plsc.store_scatter(eidb, [pos], iota + (base + v * 16), mask=m)
                return off + cum[15]

            n = lax.fori_loop(0, CHUNK // 16, bin_body, jnp.int32(0), unroll=5)
            tl = pl.ds(n, 16)
            dstb[tl] = jnp.full((16,), NPT, jnp.int32)
            srcb[tl] = jnp.zeros((16,), jnp.int32)
            eidb[tl] = jnp.zeros((16,), jnp.int32)
            lax.fori_loop(0, (n + 15) // 16, edge_mb, 0)
            return 0

        lax.fori_loop(0, NCHUNK, chunk_body, 0)

    run_pass(mb1)
    run_pass(mb2)

    def fin(nn, _):
        b64 = nn * CH
        acc = zerov
        for k in range(4):
            s = pl.ds(b64 + 16 * k, 16)
            acc = acc + anum[s] / jnp.maximum(aden[s], 1.0)
        oidx = nn * OUT_F + io7
        plsc.addupdate_scatter(outb, [oidx], acc, mask=iota < 8)
        plsc.addupdate_scatter(outb, [oidx], acc, mask=iota >= 8)
        return 0

    lax.fori_loop(0, cnt, fin, 0)

    @pl.when(wid < NW - 1)
    def _():
        pltpu.sync_copy(outb.at[pl.ds(0, NPT * OUT_F)],
                        out_hbm.at[pl.ds(lo * OUT_F, NPT * OUT_F)])

    @pl.when(wid == NW - 1)
    def _():
        pltpu.sync_copy(outb.at[pl.ds(0, LAST_CNT * OUT_F)],
                        out_hbm.at[pl.ds(lo * OUT_F, LAST_CNT * OUT_F)])


def _sc_call(wa, wm, featp, src, dst):
    kern = pl.kernel(
        _sc_body,
        out_type=jax.ShapeDtypeStruct((N_NODES * OUT_F,), jnp.float32),
        mesh=plsc.VectorSubcoreMesh(core_axis_name="c", subcore_axis_name="s",
                                    num_cores=2, num_subcores=16),
        scratch_types=[
            pltpu.VMEM(((NPT + 1) * CH,), jnp.float32),
            pltpu.VMEM(((NPT + 1) * CH,), jnp.float32),
            pltpu.VMEM(((NPT + 1) * CH,), jnp.float32),
            pltpu.VMEM((NPT * OUT_F + 32,), jnp.float32),
            pltpu.VMEM((CHUNK,), jnp.int32),
            pltpu.VMEM((CHUNK,), jnp.int32),
            pltpu.VMEM((CHUNK + 16,), jnp.int32),
            pltpu.VMEM((CHUNK + 16,), jnp.int32),
            pltpu.VMEM((CHUNK + 16,), jnp.int32),
            pltpu.VMEM((16, CH), jnp.float32),
            pltpu.VMEM((16, CH), jnp.float32),
            pltpu.VMEM((16, D_EDGE), jnp.float32),
            pltpu.SemaphoreType.DMA,
            pltpu.SemaphoreType.DMA,
            pltpu.SemaphoreType.DMA,
        ],
        compiler_params=pltpu.CompilerParams(
            needs_layout_passes=False, use_tc_tiling_on_sc=False),
    )
    return kern(wa, wm, featp, src, dst)


def kernel(feat, efeat, W_msg, b_msg, W_attn, b_attn, edge_index):
    wm, wa = _edge_mats(efeat, W_msg, b_msg, W_attn, b_attn)
    featp = jnp.pad(feat, ((0, 0), (0, D_EDGE - IN_F)))
    src = edge_index[0].astype(jnp.int32)
    dst = edge_index[1].astype(jnp.int32)
    out = _sc_call(wa, wm, featp, src, dst)
    return out.reshape(N_NODES, OUT_F)


# speedup vs baseline: 55.2475x; 1.9176x over previous
import jax
import jax.numpy as jnp
from jax import lax
from jax.experimental import pallas as pl
from jax.experimental.pallas import tpu as pltpu
from jax.experimental.pallas import tpu_sc as plsc

N_NODES = 10000
N_EDGES = 160000
IN_F = 8
OUT_F = 8
D_EDGE = 16
CH = IN_F * OUT_F
NW = 32
NPT = 313
LAST_CNT = N_NODES - (NW - 1) * NPT
CHUNK = 2000
NCHUNK = N_EDGES // CHUNK
CAP = 11008
BB = 64
NEG = -3.0e38


def _mm_body(ef, wm, bm, wa, ba, om, oa):
    x = ef[...]
    om[...] = jnp.dot(x, wm[...], preferred_element_type=jnp.float32) + bm[...]
    oa[...] = jnp.dot(x, wa[...], preferred_element_type=jnp.float32) + ba[...]


def _edge_mats(efeat, W_msg, b_msg, W_attn, b_attn):
    BLK = 2000
    return pl.pallas_call(
        _mm_body,
        grid=(N_EDGES // BLK,),
        in_specs=[
            pl.BlockSpec((BLK, D_EDGE), lambda i: (i, 0)),
            pl.BlockSpec((D_EDGE, CH), lambda i: (0, 0)),
            pl.BlockSpec((1, CH), lambda i: (0, 0)),
            pl.BlockSpec((D_EDGE, CH), lambda i: (0, 0)),
            pl.BlockSpec((1, CH), lambda i: (0, 0)),
        ],
        out_specs=[
            pl.BlockSpec((BLK, CH), lambda i: (i, 0)),
            pl.BlockSpec((BLK, CH), lambda i: (i, 0)),
        ],
        out_shape=[
            jax.ShapeDtypeStruct((N_EDGES, CH), jnp.float32),
            jax.ShapeDtypeStruct((N_EDGES, CH), jnp.float32),
        ],
    )(efeat, W_msg, b_msg.reshape(1, CH), W_attn, b_attn.reshape(1, CH))


def _sc_body(wa_hbm, wm_hbm, featp_hbm, src_hbm, dst_hbm, out_hbm,
             amax, aden, anum, outb, dstcA, srccA, dstcB, srccB,
             ce, cs, cd,
             waA, wmA, hA, waB, wmB, hB,
             sem_cA, sem_cB, sem_aA, sem_mA, sem_hA, sem_aB, sem_mB, sem_hB):
    wid = lax.axis_index("s") * 2 + lax.axis_index("c")
    lo = wid * NPT
    cnt = jnp.minimum(N_NODES - lo, NPT)
    iota = lax.iota(jnp.int32, 16)
    lane8 = (iota >= 8).astype(jnp.int32)
    io7 = iota & 7
    negv = jnp.full((16,), NEG, jnp.float32)
    zerov = jnp.zeros((16,), jnp.float32)
    padv = jnp.full((16,), NPT, jnp.int32)
    zeroiv = jnp.zeros((16,), jnp.int32)

    def init_acc(i, _):
        s = pl.ds(i * 16, 16)
        amax[s] = negv
        aden[s] = zerov
        anum[s] = zerov
        return 0

    lax.fori_loop(0, (NPT + 1) * CH // 16, init_acc, 0)

    def init_out(i, _):
        outb[pl.ds(i * 16, 16)] = zerov
        return 0

    lax.fori_loop(0, (NPT * OUT_F + 32) // 16, init_out, 0)

    def fire_chunk(ci, dbuf, sbuf, sem):
        base = ci * CHUNK
        pltpu.async_copy(dst_hbm.at[pl.ds(base, CHUNK)], dbuf, sem)
        pltpu.async_copy(src_hbm.at[pl.ds(base, CHUNK)], sbuf, sem)

    def drain_chunk(dbuf, sbuf, sem):
        pltpu.make_async_copy(dst_hbm.at[pl.ds(0, CHUNK)], dbuf, sem).wait()
        pltpu.make_async_copy(src_hbm.at[pl.ds(0, CHUNK)], sbuf, sem).wait()

    def scan_chunk(ci, off, dbuf, sbuf):
        base = ci * CHUNK

        def bin_body(v, off):
            d = dbuf[pl.ds(v * 16, 16)]
            m = (d >= lo) & (d < lo + cnt)
            cum = plsc.cumsum(m.astype(jnp.int32))
            pos = off + cum - 1
            posc = jnp.where(pos < CAP, pos, CAP + BB + iota)
            plsc.store_scatter(cd, [posc], d - lo, mask=m)
            plsc.store_scatter(cs, [posc], sbuf[pl.ds(v * 16, 16)], mask=m)
            plsc.store_scatter(ce, [posc], iota + (base + v * 16), mask=m)
            return off + cum[15]

        return lax.fori_loop(0, CHUNK // 16, bin_body, off, unroll=5)

    fire_chunk(0, dstcA, srccA, sem_cA)

    def chunk_loop(ci, off):
        even = (ci % 2) == 0

        def do(dbuf, sbuf, sem, ndbuf, nsbuf, nsem):
            @pl.when(ci + 1 < NCHUNK)
            def _():
                fire_chunk(ci + 1, ndbuf, nsbuf, nsem)
            drain_chunk(dbuf, sbuf, sem)
            return scan_chunk(ci, off, dbuf, sbuf)

        return lax.cond(
            even,
            lambda: do(dstcA, srccA, sem_cA, dstcB, srccB, sem_cB),
            lambda: do(dstcB, srccB, sem_cB, dstcA, srccA, sem_cA),
        )

    total = lax.fori_loop(0, NCHUNK, chunk_loop, jnp.int32(0))
    tcap = jnp.minimum(total, CAP)
    for t in range(4):
        s = pl.ds(tcap + 16 * t, 16)
        cd[s] = padv
        cs[s] = zeroiv
        ce[s] = zeroiv

    def fire_batch(t, wabuf, hbuf, sema, semh, wmbuf, semm, want_wm):
        eb = ce.at[pl.ds(t * BB, BB)]
        sb = cs.at[pl.ds(t * BB, BB)]
        pltpu.async_copy(wa_hbm.at[eb], wabuf, sema)
        pltpu.async_copy(featp_hbm.at[sb], hbuf, semh)
        if want_wm:
            pltpu.async_copy(wm_hbm.at[eb], wmbuf, semm)

    def drain_batch(wabuf, hbuf, sema, semh, wmbuf, semm, want_wm):
        eb0 = ce.at[pl.ds(0, BB)]
        sb0 = cs.at[pl.ds(0, BB)]
        pltpu.make_async_copy(wa_hbm.at[eb0], wabuf, sema).wait()
        pltpu.make_async_copy(featp_hbm.at[sb0], hbuf, semh).wait()
        if want_wm:
            pltpu.make_async_copy(wm_hbm.at[eb0], wmbuf, semm).wait()

    def proc_batch1(t, wabuf, hbuf):
        def edge1(j, _):
            dl = plsc.load_gather(cd, [jnp.full((16,), t * BB + j, jnp.int32)])
            base64 = dl * CH
            js = jnp.full((16,), j, jnp.int32)
            for k in range(4):
                hk = plsc.load_gather(hbuf, [js, lane8 + 2 * k])
                wak = plsc.load_gather(wabuf, [js, iota + 16 * k])
                idxk = base64 + (iota + 16 * k)
                mk = plsc.load_gather(amax, [idxk])
                plsc.store_scatter(amax, [idxk], jnp.maximum(mk, wak * hk))
            return 0

        lax.fori_loop(0, BB, edge1, 0, unroll=8)

    def proc_batch2(t, wabuf, wmbuf, hbuf):
        def edge2(j, _):
            dl = plsc.load_gather(cd, [jnp.full((16,), t * BB + j, jnp.int32)])
            base64 = dl * CH
            js = jnp.full((16,), j, jnp.int32)
            for k in range(4):
                hk = plsc.load_gather(hbuf, [js, lane8 + 2 * k])
                wak = plsc.load_gather(wabuf, [js, iota + 16 * k])
                wmk = plsc.load_gather(wmbuf, [js, iota + 16 * k])
                idxk = base64 + (iota + 16 * k)
                mk = plsc.load_gather(amax, [idxk])
                p = jnp.exp(wak * hk - mk)
                plsc.addupdate_scatter(aden, [idxk], p)
                plsc.addupdate_scatter(anum, [idxk], wmk * hk * p)
            return 0

        lax.fori_loop(0, BB, edge2, 0, unroll=8)

    def cached_sweep(sweep2):
        nb = (tcap + BB - 1) // BB

        @pl.when(nb > 0)
        def _():
            fire_batch(0, waA, hA, sem_aA, sem_hA, wmA, sem_mA, sweep2)

        def body(t, _):
            def do(wab, hb, sa, sh, wmb, sm, nwab, nhb, nsa, nsh, nwmb, nsm):
                @pl.when(t + 1 < nb)
                def _():
                    fire_batch(t + 1, nwab, nhb, nsa, nsh, nwmb, nsm, sweep2)
                drain_batch(wab, hb, sa, sh, wmb, sm, sweep2)
                if sweep2:
                    proc_batch2(t, wab, wmb, hb)
                else:
                    proc_batch1(t, wab, hb)
                return 0

            return lax.cond(
                (t % 2) == 0,
                lambda: do(waA, hA, sem_aA, sem_hA, wmA, sem_mA,
                           waB, hB, sem_aB, sem_hB, wmB, sem_mB),
                lambda: do(waB, hB, sem_aB, sem_hB, wmB, sem_mB,
                           waA, hA, sem_aA, sem_hA, wmA, sem_mA),
            )

        lax.fori_loop(0, nb, body, 0)

    def fb_fetch(b, want_wm):
        eb = ce.at[pl.ds(b * 16, 16)]
        sb = cs.at[pl.ds(b * 16, 16)]
        wa16 = waA.at[pl.ds(0, 16)]
        wm16 = wmA.at[pl.ds(0, 16)]
        h16 = hA.at[pl.ds(0, 16)]
        cpa = pltpu.async_copy(wa_hbm.at[eb], wa16, sem_aA)
        cph = pltpu.async_copy(featp_hbm.at[sb], h16, sem_hA)
        if want_wm:
            cpm = pltpu.async_copy(wm_hbm.at[eb], wm16, sem_mA)
        cpa.wait()
        cph.wait()
        if want_wm:
            cpm.wait()

    def fb_mb(b, _, sweep2):
        fb_fetch(b, sweep2)

        def edge(j, _):
            dl = plsc.load_gather(cd, [jnp.full((16,), b * 16 + j, jnp.int32)])
            base64 = dl * CH
            js = jnp.full((16,), j, jnp.int32)
            for k in range(4):
                hk = plsc.load_gather(hA, [js, lane8 + 2 * k])
                wak = plsc.load_gather(waA, [js, iota + 16 * k])
                idxk = base64 + (iota + 16 * k)
                if sweep2:
                    wmk = plsc.load_gather(wmA, [js, iota + 16 * k])
                    mk = plsc.load_gather(amax, [idxk])
                    p = jnp.exp(wak * hk - mk)
                    plsc.addupdate_scatter(aden, [idxk], p)
                    plsc.addupdate_scatter(anum, [idxk], wmk * hk * p)
                else:
                    mk = plsc.load_gather(amax, [idxk])
                    plsc.store_scatter(amax, [idxk], jnp.maximum(mk, wak * hk))
            return 0

        lax.fori_loop(0, 16, edge, 0, unroll=8)
        return 0

    def fb_sweep(sweep2):
        def chunk_body(ci, _):
            base = ci * CHUNK
            pltpu.sync_copy(dst_hbm.at[pl.ds(base, CHUNK)], dstcA)
            pltpu.sync_copy(src_hbm.at[pl.ds(base, CHUNK)], srccA)

            def bin_body(v, off):
                d = dstcA[pl.ds(v * 16, 16)]
                m = (d >= lo) & (d < lo + cnt)
                cum = plsc.cumsum(m.astype(jnp.int32))
                pos = off + cum - 1
                plsc.store_scatter(cd, [pos], d - lo, mask=m)
                plsc.store_scatter(cs, [pos], srccA[pl.ds(v * 16, 16)], mask=m)
                plsc.store_scatter(ce, [pos], iota + (base + v * 16), mask=m)
                return off + cum[15]

            n = lax.fori_loop(0, CHUNK // 16, bin_body, jnp.int32(0), unroll=5)
            tl = pl.ds(n, 16)
            cd[tl] = padv
            cs[tl] = zeroiv
            ce[tl] = zeroiv
            lax.fori_loop(0, (n + 15) // 16,
                          lambda b, c: fb_mb(b, c, sweep2), 0)
            return 0

        lax.fori_loop(0, NCHUNK, chunk_body, 0)

    ok = total <= CAP

    @pl.when(ok)
    def _():
        cached_sweep(False)
        cached_sweep(True)

    @pl.when(jnp.logical_not(ok))
    def _():
        fb_sweep(False)
        fb_sweep(True)

    def fin(nn, _):
        b64 = nn * CH
        acc = zerov
        for k in range(4):
            s = pl.ds(b64 + 16 * k, 16)
            acc = acc + anum[s] / jnp.maximum(aden[s], 1.0)
        oidx = nn * OUT_F + io7
        plsc.addupdate_scatter(outb, [oidx], acc, mask=iota < 8)
        plsc.addupdate_scatter(outb, [oidx], acc, mask=iota >= 8)
        return 0

    lax.fori_loop(0, cnt, fin, 0)

    @pl.when(wid < NW - 1)
    def _():
        pltpu.sync_copy(outb.at[pl.ds(0, NPT * OUT_F)],
                        out_hbm.at[pl.ds(lo * OUT_F, NPT * OUT_F)])

    @pl.when(wid == NW - 1)
    def _():
        pltpu.sync_copy(outb.at[pl.ds(0, LAST_CNT * OUT_F)],
                        out_hbm.at[pl.ds(lo * OUT_F, LAST_CNT * OUT_F)])


def _sc_call(wa, wm, featp, src, dst):
    kern = pl.kernel(
        _sc_body,
        out_type=jax.ShapeDtypeStruct((N_NODES * OUT_F,), jnp.float32),
        mesh=plsc.VectorSubcoreMesh(core_axis_name="c", subcore_axis_name="s",
                                    num_cores=2, num_subcores=16),
        scratch_types=[
            pltpu.VMEM(((NPT + 1) * CH,), jnp.float32),
            pltpu.VMEM(((NPT + 1) * CH,), jnp.float32),
            pltpu.VMEM(((NPT + 1) * CH,), jnp.float32),
            pltpu.VMEM((NPT * OUT_F + 32,), jnp.float32),
            pltpu.VMEM((CHUNK,), jnp.int32),
            pltpu.VMEM((CHUNK,), jnp.int32),
            pltpu.VMEM((CHUNK,), jnp.int32),
            pltpu.VMEM((CHUNK,), jnp.int32),
            pltpu.VMEM((CAP + 80,), jnp.int32),
            pltpu.VMEM((CAP + 80,), jnp.int32),
            pltpu.VMEM((CAP + 80,), jnp.int32),
            pltpu.VMEM((BB, CH), jnp.float32),
            pltpu.VMEM((BB, CH), jnp.float32),
            pltpu.VMEM((BB, D_EDGE), jnp.float32),
            pltpu.VMEM((BB, CH), jnp.float32),
            pltpu.VMEM((BB, CH), jnp.float32),
            pltpu.VMEM((BB, D_EDGE), jnp.float32),
            pltpu.SemaphoreType.DMA,
            pltpu.SemaphoreType.DMA,
            pltpu.SemaphoreType.DMA,
            pltpu.SemaphoreType.DMA,
            pltpu.SemaphoreType.DMA,
            pltpu.SemaphoreType.DMA,
            pltpu.SemaphoreType.DMA,
            pltpu.SemaphoreType.DMA,
        ],
        compiler_params=pltpu.CompilerParams(
            needs_layout_passes=False, use_tc_tiling_on_sc=False),
    )
    return kern(wa, wm, featp, src, dst)


def kernel(feat, efeat, W_msg, b_msg, W_attn, b_attn, edge_index):
    wm, wa = _edge_mats(efeat, W_msg, b_msg, W_attn, b_attn)
    featp = jnp.pad(feat, ((0, 0), (0, D_EDGE - IN_F)))
    src = edge_index[0].astype(jnp.int32)
    dst = edge_index[1].astype(jnp.int32)
    out = _sc_call(wa, wm, featp, src, dst)
    return out.reshape(N_NODES, OUT_F)


# speedup vs baseline: 61.5529x; 1.1141x over previous
import jax
import jax.numpy as jnp
from jax import lax
from jax.experimental import pallas as pl
from jax.experimental.pallas import tpu as pltpu
from jax.experimental.pallas import tpu_sc as plsc

N_NODES = 10000
N_EDGES = 160000
IN_F = 8
OUT_F = 8
D_EDGE = 16
CH = IN_F * OUT_F
NW = 32
NPT = 313
LAST_CNT = N_NODES - (NW - 1) * NPT
CHUNK = 2000
NCHUNK = N_EDGES // CHUNK
CAP = 11008
BB = 64
NEG = -3.0e38


def _mm_body(ef, wm, bm, wa, ba, om, oa):
    x = ef[...]
    om[...] = jnp.dot(x, wm[...], preferred_element_type=jnp.float32) + bm[...]
    oa[...] = jnp.dot(x, wa[...], preferred_element_type=jnp.float32) + ba[...]


def _edge_mats(efeat, W_msg, b_msg, W_attn, b_attn):
    BLK = 2000
    return pl.pallas_call(
        _mm_body,
        grid=(N_EDGES // BLK,),
        in_specs=[
            pl.BlockSpec((BLK, D_EDGE), lambda i: (i, 0)),
            pl.BlockSpec((D_EDGE, CH), lambda i: (0, 0)),
            pl.BlockSpec((1, CH), lambda i: (0, 0)),
            pl.BlockSpec((D_EDGE, CH), lambda i: (0, 0)),
            pl.BlockSpec((1, CH), lambda i: (0, 0)),
        ],
        out_specs=[
            pl.BlockSpec((BLK, CH), lambda i: (i, 0)),
            pl.BlockSpec((BLK, CH), lambda i: (i, 0)),
        ],
        out_shape=[
            jax.ShapeDtypeStruct((N_EDGES, CH), jnp.float32),
            jax.ShapeDtypeStruct((N_EDGES, CH), jnp.float32),
        ],
    )(efeat, W_msg, b_msg.reshape(1, CH), W_attn, b_attn.reshape(1, CH))


def _sc_body(wa_hbm, wm_hbm, featp_hbm, src_hbm, dst_hbm, out_hbm,
             amax, aden, anum, outb, dstcA, srccA, dstcB, srccB,
             ce, cs, cd,
             waA, wmA, hA, waB, wmB, hB,
             sem_cA, sem_cB, sem_aA, sem_mA, sem_hA, sem_aB, sem_mB, sem_hB):
    wid = lax.axis_index("s") * 2 + lax.axis_index("c")
    lo = wid * NPT
    cnt = jnp.minimum(N_NODES - lo, NPT)
    iota = lax.iota(jnp.int32, 16)
    lane8 = (iota >= 8).astype(jnp.int32)
    io7 = iota & 7
    negv = jnp.full((16,), NEG, jnp.float32)
    zerov = jnp.zeros((16,), jnp.float32)
    padv = jnp.full((16,), NPT, jnp.int32)
    zeroiv = jnp.zeros((16,), jnp.int32)

    def init_acc(i, _):
        s = pl.ds(i * 16, 16)
        amax[s] = negv
        aden[s] = zerov
        anum[s] = zerov
        return 0

    lax.fori_loop(0, (NPT + 1) * CH // 16, init_acc, 0)

    def init_out(i, _):
        outb[pl.ds(i * 16, 16)] = zerov
        return 0

    lax.fori_loop(0, (NPT * OUT_F + 32) // 16, init_out, 0)

    def fire_chunk(ci, dbuf, sbuf, sem):
        base = ci * CHUNK
        pltpu.async_copy(dst_hbm.at[pl.ds(base, CHUNK)], dbuf, sem)
        pltpu.async_copy(src_hbm.at[pl.ds(base, CHUNK)], sbuf, sem)

    def drain_chunk(dbuf, sbuf, sem):
        pltpu.make_async_copy(dst_hbm.at[pl.ds(0, CHUNK)], dbuf, sem).wait()
        pltpu.make_async_copy(src_hbm.at[pl.ds(0, CHUNK)], sbuf, sem).wait()

    def scan_chunk(ci, off, dbuf, sbuf):
        base = ci * CHUNK

        def bin_body(v, off):
            d = dbuf[pl.ds(v * 16, 16)]
            m = (d >= lo) & (d < lo + cnt)
            cum = plsc.cumsum(m.astype(jnp.int32))
            pos = off + cum - 1
            posc = jnp.where(pos < CAP, pos, CAP + BB + iota)
            plsc.store_scatter(cd, [posc], d - lo, mask=m)
            plsc.store_scatter(cs, [posc], sbuf[pl.ds(v * 16, 16)], mask=m)
            plsc.store_scatter(ce, [posc], iota + (base + v * 16), mask=m)
            return off + cum[15]

        return lax.fori_loop(0, CHUNK // 16, bin_body, off, unroll=5)

    fire_chunk(0, dstcA, srccA, sem_cA)

    def chunk_loop(ci, off):
        even = (ci % 2) == 0

        def do(dbuf, sbuf, sem, ndbuf, nsbuf, nsem):
            @pl.when(ci + 1 < NCHUNK)
            def _():
                fire_chunk(ci + 1, ndbuf, nsbuf, nsem)
            drain_chunk(dbuf, sbuf, sem)
            return scan_chunk(ci, off, dbuf, sbuf)

        return lax.cond(
            even,
            lambda: do(dstcA, srccA, sem_cA, dstcB, srccB, sem_cB),
            lambda: do(dstcB, srccB, sem_cB, dstcA, srccA, sem_cA),
        )

    total = lax.fori_loop(0, NCHUNK, chunk_loop, jnp.int32(0))
    tcap = jnp.minimum(total, CAP)
    for t in range(4):
        s = pl.ds(tcap + 16 * t, 16)
        cd[s] = padv
        cs[s] = zeroiv
        ce[s] = zeroiv

    def fire_batch(t, wabuf, hbuf, sema, semh, wmbuf, semm, want_wm):
        eb = ce.at[pl.ds(t * BB, BB)]
        sb = cs.at[pl.ds(t * BB, BB)]
        pltpu.async_copy(wa_hbm.at[eb], wabuf, sema)
        pltpu.async_copy(featp_hbm.at[sb], hbuf, semh)
        if want_wm:
            pltpu.async_copy(wm_hbm.at[eb], wmbuf, semm)

    def drain_batch(wabuf, hbuf, sema, semh, wmbuf, semm, want_wm):
        eb0 = ce.at[pl.ds(0, BB)]
        sb0 = cs.at[pl.ds(0, BB)]
        pltpu.make_async_copy(wa_hbm.at[eb0], wabuf, sema).wait()
        pltpu.make_async_copy(featp_hbm.at[sb0], hbuf, semh).wait()
        if want_wm:
            pltpu.make_async_copy(wm_hbm.at[eb0], wmbuf, semm).wait()

    def proc_batch(t, wabuf, wmbuf, hbuf):
        def edge(j, _):
            dl = plsc.load_gather(cd, [jnp.full((16,), t * BB + j, jnp.int32)])
            base64 = dl * CH
            js = jnp.full((16,), j, jnp.int32)
            for k in range(4):
                hk = plsc.load_gather(hbuf, [js, lane8 + 2 * k])
                wak = plsc.load_gather(wabuf, [js, iota + 16 * k])
                wmk = plsc.load_gather(wmbuf, [js, iota + 16 * k])
                idxk = base64 + (iota + 16 * k)
                e2 = wak * hk
                mold = plsc.load_gather(amax, [idxk])
                mnew = jnp.maximum(mold, e2)
                alpha = jnp.exp(mold - mnew)
                p = jnp.exp(e2 - mnew)
                dold = plsc.load_gather(aden, [idxk])
                nold = plsc.load_gather(anum, [idxk])
                plsc.store_scatter(amax, [idxk], mnew)
                plsc.store_scatter(aden, [idxk], dold * alpha + p)
                plsc.store_scatter(anum, [idxk], nold * alpha + wmk * hk * p)
            return 0

        lax.fori_loop(0, BB, edge, 0, unroll=8)

    def cached_sweep():
        nb = (tcap + BB - 1) // BB

        @pl.when(nb > 0)
        def _():
            fire_batch(0, waA, hA, sem_aA, sem_hA, wmA, sem_mA, True)

        def body(t, _):
            def do(wab, hb, sa, sh, wmb, sm, nwab, nhb, nsa, nsh, nwmb, nsm):
                @pl.when(t + 1 < nb)
                def _():
                    fire_batch(t + 1, nwab, nhb, nsa, nsh, nwmb, nsm, True)
                drain_batch(wab, hb, sa, sh, wmb, sm, True)
                proc_batch(t, wab, wmb, hb)
                return 0

            return lax.cond(
                (t % 2) == 0,
                lambda: do(waA, hA, sem_aA, sem_hA, wmA, sem_mA,
                           waB, hB, sem_aB, sem_hB, wmB, sem_mB),
                lambda: do(waB, hB, sem_aB, sem_hB, wmB, sem_mB,
                           waA, hA, sem_aA, sem_hA, wmA, sem_mA),
            )

        lax.fori_loop(0, nb, body, 0)

    def fb_fetch(b, want_wm):
        eb = ce.at[pl.ds(b * 16, 16)]
        sb = cs.at[pl.ds(b * 16, 16)]
        wa16 = waA.at[pl.ds(0, 16)]
        wm16 = wmA.at[pl.ds(0, 16)]
        h16 = hA.at[pl.ds(0, 16)]
        cpa = pltpu.async_copy(wa_hbm.at[eb], wa16, sem_aA)
        cph = pltpu.async_copy(featp_hbm.at[sb], h16, sem_hA)
        if want_wm:
            cpm = pltpu.async_copy(wm_hbm.at[eb], wm16, sem_mA)
        cpa.wait()
        cph.wait()
        if want_wm:
            cpm.wait()

    def fb_mb(b, _):
        fb_fetch(b, True)

        def edge(j, _):
            dl = plsc.load_gather(cd, [jnp.full((16,), b * 16 + j, jnp.int32)])
            base64 = dl * CH
            js = jnp.full((16,), j, jnp.int32)
            for k in range(4):
                hk = plsc.load_gather(hA, [js, lane8 + 2 * k])
                wak = plsc.load_gather(waA, [js, iota + 16 * k])
                wmk = plsc.load_gather(wmA, [js, iota + 16 * k])
                idxk = base64 + (iota + 16 * k)
                e2 = wak * hk
                mold = plsc.load_gather(amax, [idxk])
                mnew = jnp.maximum(mold, e2)
                alpha = jnp.exp(mold - mnew)
                p = jnp.exp(e2 - mnew)
                dold = plsc.load_gather(aden, [idxk])
                nold = plsc.load_gather(anum, [idxk])
                plsc.store_scatter(amax, [idxk], mnew)
                plsc.store_scatter(aden, [idxk], dold * alpha + p)
                plsc.store_scatter(anum, [idxk], nold * alpha + wmk * hk * p)
            return 0

        lax.fori_loop(0, 16, edge, 0, unroll=8)
        return 0

    def fb_sweep():
        def chunk_body(ci, _):
            base = ci * CHUNK
            pltpu.sync_copy(dst_hbm.at[pl.ds(base, CHUNK)], dstcA)
            pltpu.sync_copy(src_hbm.at[pl.ds(base, CHUNK)], srccA)

            def bin_body(v, off):
                d = dstcA[pl.ds(v * 16, 16)]
                m = (d >= lo) & (d < lo + cnt)
                cum = plsc.cumsum(m.astype(jnp.int32))
                pos = off + cum - 1
                plsc.store_scatter(cd, [pos], d - lo, mask=m)
                plsc.store_scatter(cs, [pos], srccA[pl.ds(v * 16, 16)], mask=m)
                plsc.store_scatter(ce, [pos], iota + (base + v * 16), mask=m)
                return off + cum[15]

            n = lax.fori_loop(0, CHUNK // 16, bin_body, jnp.int32(0), unroll=5)
            tl = pl.ds(n, 16)
            cd[tl] = padv
            cs[tl] = zeroiv
            ce[tl] = zeroiv
            lax.fori_loop(0, (n + 15) // 16, fb_mb, 0)
            return 0

        lax.fori_loop(0, NCHUNK, chunk_body, 0)

    ok = total <= CAP

    @pl.when(ok)
    def _():
        cached_sweep()

    @pl.when(jnp.logical_not(ok))
    def _():
        fb_sweep()

    def fin(nn, _):
        b64 = nn * CH
        acc = zerov
        for k in range(4):
            s = pl.ds(b64 + 16 * k, 16)
            acc = acc + anum[s] / jnp.maximum(aden[s], 1.0)
        oidx = nn * OUT_F + io7
        plsc.addupdate_scatter(outb, [oidx], acc, mask=iota < 8)
        plsc.addupdate_scatter(outb, [oidx], acc, mask=iota >= 8)
        return 0

    lax.fori_loop(0, cnt, fin, 0)

    @pl.when(wid < NW - 1)
    def _():
        pltpu.sync_copy(outb.at[pl.ds(0, NPT * OUT_F)],
                        out_hbm.at[pl.ds(lo * OUT_F, NPT * OUT_F)])

    @pl.when(wid == NW - 1)
    def _():
        pltpu.sync_copy(outb.at[pl.ds(0, LAST_CNT * OUT_F)],
                        out_hbm.at[pl.ds(lo * OUT_F, LAST_CNT * OUT_F)])


def _sc_call(wa, wm, featp, src, dst):
    kern = pl.kernel(
        _sc_body,
        out_type=jax.ShapeDtypeStruct((N_NODES * OUT_F,), jnp.float32),
        mesh=plsc.VectorSubcoreMesh(core_axis_name="c", subcore_axis_name="s",
                                    num_cores=2, num_subcores=16),
        scratch_types=[
            pltpu.VMEM(((NPT + 1) * CH,), jnp.float32),
            pltpu.VMEM(((NPT + 1) * CH,), jnp.float32),
            pltpu.VMEM(((NPT + 1) * CH,), jnp.float32),
            pltpu.VMEM((NPT * OUT_F + 32,), jnp.float32),
            pltpu.VMEM((CHUNK,), jnp.int32),
            pltpu.VMEM((CHUNK,), jnp.int32),
            pltpu.VMEM((CHUNK,), jnp.int32),
            pltpu.VMEM((CHUNK,), jnp.int32),
            pltpu.VMEM((CAP + 80,), jnp.int32),
            pltpu.VMEM((CAP + 80,), jnp.int32),
            pltpu.VMEM((CAP + 80,), jnp.int32),
            pltpu.VMEM((BB, CH), jnp.float32),
            pltpu.VMEM((BB, CH), jnp.float32),
            pltpu.VMEM((BB, D_EDGE), jnp.float32),
            pltpu.VMEM((BB, CH), jnp.float32),
            pltpu.VMEM((BB, CH), jnp.float32),
            pltpu.VMEM((BB, D_EDGE), jnp.float32),
            pltpu.SemaphoreType.DMA,
            pltpu.SemaphoreType.DMA,
            pltpu.SemaphoreType.DMA,
            pltpu.SemaphoreType.DMA,
            pltpu.SemaphoreType.DMA,
            pltpu.SemaphoreType.DMA,
            pltpu.SemaphoreType.DMA,
            pltpu.SemaphoreType.DMA,
        ],
        compiler_params=pltpu.CompilerParams(
            needs_layout_passes=False, use_tc_tiling_on_sc=False),
    )
    return kern(wa, wm, featp, src, dst)


def kernel(feat, efeat, W_msg, b_msg, W_attn, b_attn, edge_index):
    wm, wa = _edge_mats(efeat, W_msg, b_msg, W_attn, b_attn)
    featp = jnp.pad(feat, ((0, 0), (0, D_EDGE - IN_F)))
    src = edge_index[0].astype(jnp.int32)
    dst = edge_index[1].astype(jnp.int32)
    out = _sc_call(wa, wm, featp, src, dst)
    return out.reshape(N_NODES, OUT_F)


# speedup vs baseline: 72.1202x; 1.1717x over previous
import jax
import jax.numpy as jnp
from jax import lax
from jax.experimental import pallas as pl
from jax.experimental.pallas import tpu as pltpu
from jax.experimental.pallas import tpu_sc as plsc

N_NODES = 10000
N_EDGES = 160000
IN_F = 8
OUT_F = 8
D_EDGE = 16
CH = IN_F * OUT_F
NW = 32
NPT = 313
LAST_CNT = N_NODES - (NW - 1) * NPT
CHUNK = 2000
NCHUNK = N_EDGES // CHUNK
CAP = 11008
BB = 64
NEG = -3.0e38


def _mm_body(ef, w, b, o):
    o[...] = jnp.dot(ef[...], w[...],
                     preferred_element_type=jnp.float32) + b[...]


def _edge_mats(efeat, W_cat, b_cat):
    BLK = 2000
    return pl.pallas_call(
        _mm_body,
        grid=(N_EDGES // BLK,),
        in_specs=[
            pl.BlockSpec((BLK, D_EDGE), lambda i: (i, 0)),
            pl.BlockSpec((D_EDGE, 2 * CH), lambda i: (0, 0)),
            pl.BlockSpec((1, 2 * CH), lambda i: (0, 0)),
        ],
        out_specs=pl.BlockSpec((BLK, 2 * CH), lambda i: (i, 0)),
        out_shape=jax.ShapeDtypeStruct((N_EDGES, 2 * CH), jnp.float32),
    )(efeat, W_cat, b_cat.reshape(1, 2 * CH))


def _sc_body(wawm_hbm, featp_hbm, src_hbm, dst_hbm, out_hbm,
             amax, aden, anum, outb, dstcA, srccA, dstcB, srccB,
             ce, cs, cd,
             wbA, hA, wbB, hB,
             sem_cA, sem_cB, sem_aA, sem_hA, sem_aB, sem_hB):
    wid = lax.axis_index("s") * 2 + lax.axis_index("c")
    lo = wid * NPT
    cnt = jnp.minimum(N_NODES - lo, NPT)
    iota = lax.iota(jnp.int32, 16)
    lane8 = (iota >= 8).astype(jnp.int32)
    io7 = iota & 7
    negv = jnp.full((16,), NEG, jnp.float32)
    zerov = jnp.zeros((16,), jnp.float32)
    padv = jnp.full((16,), NPT, jnp.int32)
    zeroiv = jnp.zeros((16,), jnp.int32)

    def init_acc(i, _):
        s = pl.ds(i * 16, 16)
        amax[s] = negv
        aden[s] = zerov
        anum[s] = zerov
        return 0

    lax.fori_loop(0, (NPT + 1) * CH // 16, init_acc, 0)

    def init_out(i, _):
        outb[pl.ds(i * 16, 16)] = zerov
        return 0

    lax.fori_loop(0, (NPT * OUT_F + 32) // 16, init_out, 0)

    def fire_chunk(ci, dbuf, sbuf, sem):
        base = ci * CHUNK
        pltpu.async_copy(dst_hbm.at[pl.ds(base, CHUNK)], dbuf, sem)
        pltpu.async_copy(src_hbm.at[pl.ds(base, CHUNK)], sbuf, sem)

    def drain_chunk(dbuf, sbuf, sem):
        pltpu.make_async_copy(dst_hbm.at[pl.ds(0, CHUNK)], dbuf, sem).wait()
        pltpu.make_async_copy(src_hbm.at[pl.ds(0, CHUNK)], sbuf, sem).wait()

    def scan_chunk(ci, off, dbuf, sbuf):
        base = ci * CHUNK

        def bin_body(v, off):
            d = dbuf[pl.ds(v * 16, 16)]
            m = (d >= lo) & (d < lo + cnt)
            cum = plsc.cumsum(m.astype(jnp.int32))
            pos = off + cum - 1
            posc = jnp.where(pos < CAP, pos, CAP + BB + iota)
            plsc.store_scatter(cd, [posc], d - lo, mask=m)
            plsc.store_scatter(cs, [posc], sbuf[pl.ds(v * 16, 16)], mask=m)
            plsc.store_scatter(ce, [posc], iota + (base + v * 16), mask=m)
            return off + cum[15]

        return lax.fori_loop(0, CHUNK // 16, bin_body, off, unroll=5)

    fire_chunk(0, dstcA, srccA, sem_cA)

    def chunk_loop(ci, off):
        even = (ci % 2) == 0

        def do(dbuf, sbuf, sem, ndbuf, nsbuf, nsem):
            @pl.when(ci + 1 < NCHUNK)
            def _():
                fire_chunk(ci + 1, ndbuf, nsbuf, nsem)
            drain_chunk(dbuf, sbuf, sem)
            return scan_chunk(ci, off, dbuf, sbuf)

        return lax.cond(
            even,
            lambda: do(dstcA, srccA, sem_cA, dstcB, srccB, sem_cB),
            lambda: do(dstcB, srccB, sem_cB, dstcA, srccA, sem_cA),
        )

    total = lax.fori_loop(0, NCHUNK, chunk_loop, jnp.int32(0))
    tcap = jnp.minimum(total, CAP)
    for t in range(4):
        s = pl.ds(tcap + 16 * t, 16)
        cd[s] = padv
        cs[s] = zeroiv
        ce[s] = zeroiv

    def fire_batch(t, wbuf, hbuf, sema, semh):
        eb = ce.at[pl.ds(t * BB, BB)]
        sb = cs.at[pl.ds(t * BB, BB)]
        pltpu.async_copy(wawm_hbm.at[eb], wbuf, sema)
        pltpu.async_copy(featp_hbm.at[sb], hbuf, semh)

    def drain_batch(wbuf, hbuf, sema, semh):
        eb0 = ce.at[pl.ds(0, BB)]
        sb0 = cs.at[pl.ds(0, BB)]
        pltpu.make_async_copy(wawm_hbm.at[eb0], wbuf, sema).wait()
        pltpu.make_async_copy(featp_hbm.at[sb0], hbuf, semh).wait()

    def proc_batch(t, wbuf, hbuf):
        def edge(j, _):
            dl = plsc.load_gather(cd, [jnp.full((16,), t * BB + j, jnp.int32)])
            base64 = dl * CH
            js = jnp.full((16,), j, jnp.int32)
            for k in range(4):
                hk = plsc.load_gather(hbuf, [js, lane8 + 2 * k])
                wak = plsc.load_gather(wbuf, [js, iota + 16 * k])
                wmk = plsc.load_gather(wbuf, [js, CH + iota + 16 * k])
                idxk = base64 + (iota + 16 * k)
                e2 = wak * hk
                mold = plsc.load_gather(amax, [idxk])
                mnew = jnp.maximum(mold, e2)
                alpha = jnp.exp(mold - mnew)
                p = jnp.exp(e2 - mnew)
                dold = plsc.load_gather(aden, [idxk])
                nold = plsc.load_gather(anum, [idxk])
                plsc.store_scatter(amax, [idxk], mnew)
                plsc.store_scatter(aden, [idxk], dold * alpha + p)
                plsc.store_scatter(anum, [idxk], nold * alpha + wmk * hk * p)
            return 0

        lax.fori_loop(0, BB, edge, 0, unroll=8)

    def cached_sweep():
        nb = (tcap + BB - 1) // BB

        @pl.when(nb > 0)
        def _():
            fire_batch(0, wbA, hA, sem_aA, sem_hA)

        def body(t, _):
            def do(wb, hb, sa, sh, nwb, nhb, nsa, nsh):
                @pl.when(t + 1 < nb)
                def _():
                    fire_batch(t + 1, nwb, nhb, nsa, nsh)
                drain_batch(wb, hb, sa, sh)
                proc_batch(t, wb, hb)
                return 0

            return lax.cond(
                (t % 2) == 0,
                lambda: do(wbA, hA, sem_aA, sem_hA, wbB, hB, sem_aB, sem_hB),
                lambda: do(wbB, hB, sem_aB, sem_hB, wbA, hA, sem_aA, sem_hA),
            )

        lax.fori_loop(0, nb, body, 0)

    def fb_fetch(b):
        eb = ce.at[pl.ds(b * 16, 16)]
        sb = cs.at[pl.ds(b * 16, 16)]
        wb16 = wbA.at[pl.ds(0, 16)]
        h16 = hA.at[pl.ds(0, 16)]
        cpa = pltpu.async_copy(wawm_hbm.at[eb], wb16, sem_aA)
        cph = pltpu.async_copy(featp_hbm.at[sb], h16, sem_hA)
        cpa.wait()
        cph.wait()

    def fb_mb(b, _):
        fb_fetch(b)

        def edge(j, _):
            dl = plsc.load_gather(cd, [jnp.full((16,), b * 16 + j, jnp.int32)])
            base64 = dl * CH
            js = jnp.full((16,), j, jnp.int32)
            for k in range(4):
                hk = plsc.load_gather(hA, [js, lane8 + 2 * k])
                wak = plsc.load_gather(wbA, [js, iota + 16 * k])
                wmk = plsc.load_gather(wbA, [js, CH + iota + 16 * k])
                idxk = base64 + (iota + 16 * k)
                e2 = wak * hk
                mold = plsc.load_gather(amax, [idxk])
                mnew = jnp.maximum(mold, e2)
                alpha = jnp.exp(mold - mnew)
                p = jnp.exp(e2 - mnew)
                dold = plsc.load_gather(aden, [idxk])
                nold = plsc.load_gather(anum, [idxk])
                plsc.store_scatter(amax, [idxk], mnew)
                plsc.store_scatter(aden, [idxk], dold * alpha + p)
                plsc.store_scatter(anum, [idxk], nold * alpha + wmk * hk * p)
            return 0

        lax.fori_loop(0, 16, edge, 0, unroll=8)
        return 0

    def fb_sweep():
        def chunk_body(ci, _):
            base = ci * CHUNK
            pltpu.sync_copy(dst_hbm.at[pl.ds(base, CHUNK)], dstcA)
            pltpu.sync_copy(src_hbm.at[pl.ds(base, CHUNK)], srccA)

            def bin_body(v, off):
                d = dstcA[pl.ds(v * 16, 16)]
                m = (d >= lo) & (d < lo + cnt)
                cum = plsc.cumsum(m.astype(jnp.int32))
                pos = off + cum - 1
                plsc.store_scatter(cd, [pos], d - lo, mask=m)
                plsc.store_scatter(cs, [pos], srccA[pl.ds(v * 16, 16)], mask=m)
                plsc.store_scatter(ce, [pos], iota + (base + v * 16), mask=m)
                return off + cum[15]

            n = lax.fori_loop(0, CHUNK // 16, bin_body, jnp.int32(0), unroll=5)
            tl = pl.ds(n, 16)
            cd[tl] = padv
            cs[tl] = zeroiv
            ce[tl] = zeroiv
            lax.fori_loop(0, (n + 15) // 16, fb_mb, 0)
            return 0

        lax.fori_loop(0, NCHUNK, chunk_body, 0)

    ok = total <= CAP

    @pl.when(ok)
    def _():
        cached_sweep()

    @pl.when(jnp.logical_not(ok))
    def _():
        fb_sweep()

    def fin(nn, _):
        b64 = nn * CH
        acc = zerov
        for k in range(4):
            s = pl.ds(b64 + 16 * k, 16)
            acc = acc + anum[s] / jnp.maximum(aden[s], 1.0)
        oidx = nn * OUT_F + io7
        plsc.addupdate_scatter(outb, [oidx], acc, mask=iota < 8)
        plsc.addupdate_scatter(outb, [oidx], acc, mask=iota >= 8)
        return 0

    lax.fori_loop(0, cnt, fin, 0)

    @pl.when(wid < NW - 1)
    def _():
        pltpu.sync_copy(outb.at[pl.ds(0, NPT * OUT_F)],
                        out_hbm.at[pl.ds(lo * OUT_F, NPT * OUT_F)])

    @pl.when(wid == NW - 1)
    def _():
        pltpu.sync_copy(outb.at[pl.ds(0, LAST_CNT * OUT_F)],
                        out_hbm.at[pl.ds(lo * OUT_F, LAST_CNT * OUT_F)])


def _sc_call(wawm, featp, src, dst):
    kern = pl.kernel(
        _sc_body,
        out_type=jax.ShapeDtypeStruct((N_NODES * OUT_F,), jnp.float32),
        mesh=plsc.VectorSubcoreMesh(core_axis_name="c", subcore_axis_name="s",
                                    num_cores=2, num_subcores=16),
        scratch_types=[
            pltpu.VMEM(((NPT + 1) * CH,), jnp.float32),
            pltpu.VMEM(((NPT + 1) * CH,), jnp.float32),
            pltpu.VMEM(((NPT + 1) * CH,), jnp.float32),
            pltpu.VMEM((NPT * OUT_F + 32,), jnp.float32),
            pltpu.VMEM((CHUNK,), jnp.int32),
            pltpu.VMEM((CHUNK,), jnp.int32),
            pltpu.VMEM((CHUNK,), jnp.int32),
            pltpu.VMEM((CHUNK,), jnp.int32),
            pltpu.VMEM((CAP + 80,), jnp.int32),
            pltpu.VMEM((CAP + 80,), jnp.int32),
            pltpu.VMEM((CAP + 80,), jnp.int32),
            pltpu.VMEM((BB, 2 * CH), jnp.float32),
            pltpu.VMEM((BB, D_EDGE), jnp.float32),
            pltpu.VMEM((BB, 2 * CH), jnp.float32),
            pltpu.VMEM((BB, D_EDGE), jnp.float32),
            pltpu.SemaphoreType.DMA,
            pltpu.SemaphoreType.DMA,
            pltpu.SemaphoreType.DMA,
            pltpu.SemaphoreType.DMA,
            pltpu.SemaphoreType.DMA,
            pltpu.SemaphoreType.DMA,
        ],
        compiler_params=pltpu.CompilerParams(
            needs_layout_passes=False, use_tc_tiling_on_sc=False),
    )
    return kern(wawm, featp, src, dst)


def kernel(feat, efeat, W_msg, b_msg, W_attn, b_attn, edge_index):
    W_cat = jnp.concatenate([W_attn, W_msg], axis=1)
    b_cat = jnp.concatenate([b_attn, b_msg], axis=0)
    wawm = _edge_mats(efeat, W_cat, b_cat)
    featp = jnp.pad(feat, ((0, 0), (0, D_EDGE - IN_F)))
    src = edge_index[0].astype(jnp.int32)
    dst = edge_index[1].astype(jnp.int32)
    out = _sc_call(wawm, featp, src, dst)
    return out.reshape(N_NODES, OUT_F)
